# baseline (speedup 1.0000x reference)
import jax
import jax.numpy as jnp
from jax.experimental import pallas as pl
from jax.experimental.pallas import tpu as pltpu

B, N, D, E = 2, 2048, 768, 8
TOPK = 2


def _moe_block_kernel(x_ref, gate_ref, w_ref, b_ref, o_ref):
    xb = x_ref[...]
    g = jnp.dot(xb, gate_ref[...], preferred_element_type=jnp.float32)
    g = g - jnp.max(g, axis=1, keepdims=True)
    p = jnp.exp(g)
    p = p / jnp.sum(p, axis=1, keepdims=True)
    col = jax.lax.broadcasted_iota(jnp.int32, p.shape, 1)
    m1 = jnp.max(p, axis=1, keepdims=True)
    i1 = jnp.min(jnp.where(p == m1, col, E), axis=1, keepdims=True)
    p2 = jnp.where(col == i1, -1.0, p)
    m2 = jnp.max(p2, axis=1, keepdims=True)
    i2 = jnp.min(jnp.where(p2 == m2, col, E), axis=1, keepdims=True)
    sel = (col == i1) | (col == i2)
    gsel = jnp.where(sel, p, 0.0)
    denom = jnp.maximum(jnp.sum(gsel, axis=1, keepdims=True), 1e-12)
    w = gsel / denom
    acc = jnp.dot(w, b_ref[...], preferred_element_type=jnp.float32)
    for e in range(E):
        y = jnp.dot(xb, w_ref[e], preferred_element_type=jnp.float32)
        acc = acc + w[:, e:e + 1] * y
    o_ref[...] = acc


def kernel(x, gate, W_e, b_e):
    BN = B * N
    T = 2048
    xf = x.reshape(BN, D)
    out = pl.pallas_call(
        _moe_block_kernel,
        grid=(BN // T,),
        in_specs=[
            pl.BlockSpec((T, D), lambda i: (i, 0)),
            pl.BlockSpec((D, E), lambda i: (0, 0)),
            pl.BlockSpec((E, D, D), lambda i: (0, 0, 0)),
            pl.BlockSpec((E, D), lambda i: (0, 0)),
        ],
        out_specs=pl.BlockSpec((T, D), lambda i: (i, 0)),
        out_shape=jax.ShapeDtypeStruct((BN, D), jnp.float32),
        compiler_params=pltpu.CompilerParams(
            dimension_semantics=("arbitrary",),
        ),
    )(xf, gate, W_e, b_e)
    return out.reshape(B, N, D)


# speedup vs baseline: 1.0330x; 1.0330x over previous
import jax
import jax.numpy as jnp
from jax.experimental import pallas as pl
from jax.experimental.pallas import tpu as pltpu

B, N, D, E = 2, 2048, 768, 8
TOPK = 2


def _moe_block_kernel(x_ref, gate_ref, w_ref, b_ref, o_ref):
    xb = x_ref[...]
    g = jnp.dot(xb, gate_ref[...], preferred_element_type=jnp.float32)
    g = g - jnp.max(g, axis=1, keepdims=True)
    p = jnp.exp(g)
    p = p / jnp.sum(p, axis=1, keepdims=True)
    col = jax.lax.broadcasted_iota(jnp.int32, p.shape, 1)
    m1 = jnp.max(p, axis=1, keepdims=True)
    i1 = jnp.min(jnp.where(p == m1, col, E), axis=1, keepdims=True)
    p2 = jnp.where(col == i1, -1.0, p)
    m2 = jnp.max(p2, axis=1, keepdims=True)
    i2 = jnp.min(jnp.where(p2 == m2, col, E), axis=1, keepdims=True)
    sel = (col == i1) | (col == i2)
    gsel = jnp.where(sel, p, 0.0)
    denom = jnp.maximum(jnp.sum(gsel, axis=1, keepdims=True), 1e-12)
    w = gsel / denom
    acc = jnp.dot(w, b_ref[...], preferred_element_type=jnp.float32)
    for e in range(E):
        y = jnp.dot(xb, w_ref[e], preferred_element_type=jnp.float32)
        acc = acc + w[:, e:e + 1] * y
    o_ref[...] = acc


def kernel(x, gate, W_e, b_e):
    BN = B * N
    T = 1024
    xf = x.reshape(BN, D)
    out = pl.pallas_call(
        _moe_block_kernel,
        grid=(BN // T,),
        in_specs=[
            pl.BlockSpec((T, D), lambda i: (i, 0)),
            pl.BlockSpec((D, E), lambda i: (0, 0)),
            pl.BlockSpec((E, D, D), lambda i: (0, 0, 0)),
            pl.BlockSpec((E, D), lambda i: (0, 0)),
        ],
        out_specs=pl.BlockSpec((T, D), lambda i: (i, 0)),
        out_shape=jax.ShapeDtypeStruct((BN, D), jnp.float32),
        compiler_params=pltpu.CompilerParams(
            dimension_semantics=("arbitrary",),
        ),
    )(xf, gate, W_e, b_e)
    return out.reshape(B, N, D)
